# baseline (speedup 1.0000x reference)
import functools

import jax
import jax.numpy as jnp
from jax import lax
from jax.experimental import pallas as pl
from jax.experimental.pallas import tpu as pltpu
from jax.experimental.pallas import tpu_sc as plsc

B, S = 4, 2048
T = B * S
D = 768
E = 8
TB = 1024
NB_A = T // TB
BLK = 512
PAD = T + E * BLK
NB_C = PAD // BLK


def _gating_body(x_ref, gw_ref, gb_ref, idx_ref, rank_ref, cnt_ref, var_ref,
                 cnt_s, sum_s, sq_s, tri_s):
    i = pl.program_id(0)

    @pl.when(i == 0)
    def _():
        cnt_s[...] = jnp.zeros_like(cnt_s)
        sum_s[...] = jnp.zeros_like(sum_s)
        sq_s[...] = jnp.zeros_like(sq_s)
        tr = jax.lax.broadcasted_iota(jnp.int32, (TB, TB), 0)
        tc = jax.lax.broadcasted_iota(jnp.int32, (TB, TB), 1)
        tri_s[...] = (tr <= tc).astype(jnp.float32)

    x = x_ref[...]
    gw = gw_ref[...]
    logits = jax.lax.dot_general(gw, x, (((1,), (1,)), ((), ())),
                                 preferred_element_type=jnp.float32)
    logits = logits + gb_ref[...][:, 0:1]
    m = jnp.max(logits, axis=0, keepdims=True)
    ex = jnp.exp(logits - m)
    s = ex / jnp.sum(ex, axis=0, keepdims=True)

    smax = jnp.max(s, axis=0, keepdims=True)
    eiota = jax.lax.broadcasted_iota(jnp.int32, (E, TB), 0)
    idx = jnp.min(jnp.where(s >= smax, eiota, E), axis=0, keepdims=True)
    onehot = (eiota == idx).astype(jnp.float32)
    incl = jax.lax.dot_general(onehot, tri_s[...], (((1,), (0,)), ((), ())),
                               preferred_element_type=jnp.float32)
    prev = cnt_s[...][:, 0:1]
    rank = jnp.sum(onehot * (prev + incl - 1.0), axis=0, keepdims=True)

    idx_ref[...] = idx.reshape(1, 1, TB)
    rank_ref[...] = rank.astype(jnp.int32).reshape(1, 1, TB)

    newcnt = prev + incl[:, TB - 1:TB]
    cnt_s[...] = jnp.broadcast_to(newcnt, cnt_s.shape)
    sum_new = sum_s[...][:, 0:1] + jnp.sum(s, axis=1, keepdims=True)
    sq_new = sq_s[...][:, 0:1] + jnp.sum(s * s, axis=1, keepdims=True)
    sum_s[...] = jnp.broadcast_to(sum_new, sum_s.shape)
    sq_s[...] = jnp.broadcast_to(sq_new, sq_s.shape)

    nf = float(T)
    var = (sq_new - sum_new * sum_new / nf) / (nf - 1.0)
    cnt_ref[...] = jnp.broadcast_to(newcnt, cnt_ref.shape)
    var_ref[...] = jnp.broadcast_to(var, var_ref.shape)


_gating_call = pl.pallas_call(
    _gating_body,
    grid=(NB_A,),
    in_specs=[
        pl.BlockSpec((TB, D), lambda i: (i, 0)),
        pl.BlockSpec((E, D), lambda i: (0, 0)),
        pl.BlockSpec((E, 128), lambda i: (0, 0)),
    ],
    out_specs=[
        pl.BlockSpec((1, 1, TB), lambda i: (i, 0, 0)),
        pl.BlockSpec((1, 1, TB), lambda i: (i, 0, 0)),
        pl.BlockSpec((E, 128), lambda i: (0, 0)),
        pl.BlockSpec((E, 128), lambda i: (0, 0)),
    ],
    out_shape=[
        jax.ShapeDtypeStruct((NB_A, 1, TB), jnp.int32),
        jax.ShapeDtypeStruct((NB_A, 1, TB), jnp.int32),
        jax.ShapeDtypeStruct((E, 128), jnp.float32),
        jax.ShapeDtypeStruct((E, 128), jnp.float32),
    ],
    scratch_shapes=[
        pltpu.VMEM((E, 128), jnp.float32),
        pltpu.VMEM((E, 128), jnp.float32),
        pltpu.VMEM((E, 128), jnp.float32),
        pltpu.VMEM((TB, TB), jnp.float32),
    ],
)


_NC, _NS = 2, 16
_NW = _NC * _NS
_TPW = T // _NW
_CH = 32
_NCH = _TPW // _CH
_NBUF = 4

_sc_mesh = plsc.VectorSubcoreMesh(core_axis_name="c", subcore_axis_name="s")


@functools.partial(
    pl.kernel,
    out_type=[
        jax.ShapeDtypeStruct((PAD, D), jnp.float32),
        jax.ShapeDtypeStruct((T // _CH, _CH), jnp.int32),
    ],
    mesh=_sc_mesh,
    scratch_types=[
        pltpu.VMEM((_TPW,), jnp.int32),
        pltpu.VMEM((_TPW,), jnp.int32),
        pltpu.VMEM((16,), jnp.int32),
        pltpu.VMEM((_NCH, _CH), jnp.int32),
        pltpu.VMEM((_NBUF, _CH, D), jnp.float32),
        pltpu.SemaphoreType.DMA,
        pltpu.SemaphoreType.DMA,
    ],
    compiler_params=pltpu.CompilerParams(needs_layout_passes=False),
)
def _dispatch_call(flat_hbm, idx_hbm, rank_hbm, base_hbm, xs_hbm, pos_hbm,
                   idx_v, rank_v, base_v, pos_c, rows, sem_lin, sem_ind):
    wid = lax.axis_index("s") * _NC + lax.axis_index("c")
    t0 = wid * _TPW
    lins = [None] * _NCH
    inds = [None] * _NCH
    for j in range(_NBUF - 1):
        lins[j] = pltpu.async_copy(flat_hbm.at[pl.ds(t0 + j * _CH, _CH)],
                                   rows.at[j], sem_lin)
    pltpu.sync_copy(idx_hbm.at[pl.ds(t0, _TPW)], idx_v)
    pltpu.sync_copy(rank_hbm.at[pl.ds(t0, _TPW)], rank_v)
    pltpu.sync_copy(base_hbm, base_v)
    for c in range(_TPW // 16):
        e16 = idx_v[pl.ds(c * 16, 16)]
        b16 = plsc.load_gather(base_v, [e16])
        r16 = rank_v[pl.ds(c * 16, 16)]
        pos_c[c * 16 // _CH, pl.ds((c * 16) % _CH, 16)] = b16 + r16
    pltpu.sync_copy(pos_c, pos_hbm.at[pl.ds(wid * _NCH, _NCH)])
    for j in range(_NCH):
        k = j + _NBUF - 1
        if k < _NCH:
            if j >= 1:
                inds[j - 1].wait()
            lins[k] = pltpu.async_copy(
                flat_hbm.at[pl.ds(t0 + k * _CH, _CH)],
                rows.at[k % _NBUF], sem_lin)
        lins[j].wait()
        inds[j] = pltpu.async_copy(rows.at[j % _NBUF],
                                   xs_hbm.at[pos_c.at[j]], sem_ind)
    for j in range(max(0, _NCH - _NBUF), _NCH):
        inds[j].wait()


@functools.partial(
    pl.kernel,
    out_type=jax.ShapeDtypeStruct((T, D), jnp.float32),
    mesh=_sc_mesh,
    scratch_types=[
        pltpu.VMEM((_NCH, _CH), jnp.int32),
        pltpu.VMEM((_NBUF, _CH, D), jnp.float32),
        pltpu.SemaphoreType.DMA,
        pltpu.SemaphoreType.DMA,
    ],
)
def _combine_call(ys_hbm, pos_hbm, out_hbm, pos_c, rows, sem_g, sem_lin):
    wid = lax.axis_index("s") * _NC + lax.axis_index("c")
    t0 = wid * _TPW
    pltpu.sync_copy(pos_hbm.at[pl.ds(wid * _NCH, _NCH)], pos_c)
    gats = [None] * _NCH
    outs = [None] * _NCH
    for j in range(_NBUF - 1):
        gats[j] = pltpu.async_copy(ys_hbm.at[pos_c.at[j]], rows.at[j], sem_g)
    for j in range(_NCH):
        k = j + _NBUF - 1
        if k < _NCH:
            if j >= 1:
                outs[j - 1].wait()
            gats[k] = pltpu.async_copy(ys_hbm.at[pos_c.at[k]],
                                       rows.at[k % _NBUF], sem_g)
        gats[j].wait()
        outs[j] = pltpu.async_copy(rows.at[j % _NBUF],
                                   out_hbm.at[pl.ds(t0 + j * _CH, _CH)],
                                   sem_lin)
    for j in range(max(0, _NCH - _NBUF), _NCH):
        outs[j].wait()


def _matmul_body(be_ref, slot_ref, first_ref, pref_ref, act_ref, x_ref, w_hbm,
                 b_ref, o_ref, w2, sems):
    i = pl.program_id(0)
    slot = slot_ref[i]

    @pl.when(i == 0)
    def _():
        pltpu.make_async_copy(w_hbm.at[be_ref[0]], w2.at[0], sems.at[0]).start()

    @pl.when(pref_ref[i] == 1)
    def _():
        nslot = slot_ref[i + 1]
        pltpu.make_async_copy(w_hbm.at[be_ref[i + 1]], w2.at[nslot],
                              sems.at[nslot]).start()

    @pl.when(first_ref[i] == 1)
    def _():
        pltpu.make_async_copy(w_hbm.at[be_ref[i]], w2.at[slot],
                              sems.at[slot]).wait()

    for s in range(2):
        @pl.when(jnp.logical_and(act_ref[i] == 1, slot == s))
        def _(s=s):
            x = x_ref[...]
            o_ref[...] = jax.lax.dot_general(
                x, w2[s], (((1,), (1,)), ((), ())),
                preferred_element_type=jnp.float32) + b_ref[0]


def _act_idx(i, act):
    return jnp.where(act[i] == 1, i, NB_C - 1)


_matmul_call = pl.pallas_call(
    _matmul_body,
    grid_spec=pltpu.PrefetchScalarGridSpec(
        num_scalar_prefetch=5,
        grid=(NB_C,),
        in_specs=[
            pl.BlockSpec((BLK, D),
                         lambda i, be, sl, fi, pf, ac: (_act_idx(i, ac), 0)),
            pl.BlockSpec(memory_space=pl.ANY),
            pl.BlockSpec((1, 1, D),
                         lambda i, be, sl, fi, pf, ac: (be[i], 0, 0)),
        ],
        out_specs=pl.BlockSpec(
            (BLK, D), lambda i, be, sl, fi, pf, ac: (_act_idx(i, ac), 0)),
        scratch_shapes=[
            pltpu.VMEM((2, D, D), jnp.float32),
            pltpu.SemaphoreType.DMA((2,)),
        ],
    ),
    out_shape=jax.ShapeDtypeStruct((PAD, D), jnp.float32),
)


def kernel(sequences, expert_w, expert_b, gating_w, gating_b):
    flat = sequences.reshape(T, D)
    gb2 = jnp.broadcast_to(gating_b[:, None], (E, 128))

    idx3, rank3, cnt, var = _gating_call(flat, gating_w, gb2)
    idx = idx3.reshape(T)
    rank = rank3.reshape(T)
    variances = var[:, 0]

    counts = cnt[:, 0].astype(jnp.int32)
    padded = ((counts + BLK - 1) // BLK) * BLK
    cum = jnp.cumsum(padded)
    base = cum - padded
    blk_starts = jnp.arange(NB_C, dtype=jnp.int32) * BLK
    be = jnp.minimum(
        jnp.sum((cum[None, :] <= blk_starts[:, None]).astype(jnp.int32), axis=1),
        E - 1).astype(jnp.int32)

    base16 = jnp.concatenate([base, jnp.zeros((8,), jnp.int32)])

    nactive = cum[E - 1] // BLK
    act = (jnp.arange(NB_C, dtype=jnp.int32) < nactive).astype(jnp.int32)
    chg = (be[1:] != be[:-1]).astype(jnp.int32)
    first = jnp.concatenate([jnp.ones((1,), jnp.int32), chg]) * act
    slot = (jnp.cumsum(first) - 1) % 2
    pref = jnp.concatenate([first[1:], jnp.zeros((1,), jnp.int32)])

    x_sorted, pos = _dispatch_call(flat, idx, rank, base16)
    y_sorted = _matmul_call(be, slot, first, pref, act, x_sorted, expert_w,
                            expert_b.reshape(E, 1, D))
    out = _combine_call(y_sorted, pos)
    return (out.reshape(B, S, D), variances)


# speedup vs baseline: 1.0022x; 1.0022x over previous
import functools

import jax
import jax.numpy as jnp
from jax import lax
from jax.experimental import pallas as pl
from jax.experimental.pallas import tpu as pltpu
from jax.experimental.pallas import tpu_sc as plsc

B, S = 4, 2048
T = B * S
D = 768
E = 8
TB = 1024
NB_A = T // TB
BLK = 512
PAD = T + E * BLK
NB_C = PAD // BLK


def _gating_body(x_ref, gw_ref, gb_ref, idx_ref, rank_ref, cnt_ref, var_ref,
                 cnt_s, sum_s, sq_s, tri_s):
    i = pl.program_id(0)

    @pl.when(i == 0)
    def _():
        cnt_s[...] = jnp.zeros_like(cnt_s)
        sum_s[...] = jnp.zeros_like(sum_s)
        sq_s[...] = jnp.zeros_like(sq_s)
        tr = jax.lax.broadcasted_iota(jnp.int32, (TB, TB), 0)
        tc = jax.lax.broadcasted_iota(jnp.int32, (TB, TB), 1)
        tri_s[...] = (tr <= tc).astype(jnp.float32)

    x = x_ref[...]
    gw = gw_ref[...]
    logits = jax.lax.dot_general(gw, x, (((1,), (1,)), ((), ())),
                                 preferred_element_type=jnp.float32)
    logits = logits + gb_ref[...][:, 0:1]
    m = jnp.max(logits, axis=0, keepdims=True)
    ex = jnp.exp(logits - m)
    s = ex / jnp.sum(ex, axis=0, keepdims=True)

    smax = jnp.max(s, axis=0, keepdims=True)
    eiota = jax.lax.broadcasted_iota(jnp.int32, (E, TB), 0)
    idx = jnp.min(jnp.where(s >= smax, eiota, E), axis=0, keepdims=True)
    onehot = (eiota == idx).astype(jnp.float32)
    incl = jax.lax.dot_general(onehot, tri_s[...], (((1,), (0,)), ((), ())),
                               preferred_element_type=jnp.float32)
    prev = cnt_s[...][:, 0:1]
    rank = jnp.sum(onehot * (prev + incl - 1.0), axis=0, keepdims=True)

    idx_ref[...] = idx.reshape(1, 1, TB)
    rank_ref[...] = rank.astype(jnp.int32).reshape(1, 1, TB)

    newcnt = prev + incl[:, TB - 1:TB]
    cnt_s[...] = jnp.broadcast_to(newcnt, cnt_s.shape)
    sum_new = sum_s[...][:, 0:1] + jnp.sum(s, axis=1, keepdims=True)
    sq_new = sq_s[...][:, 0:1] + jnp.sum(s * s, axis=1, keepdims=True)
    sum_s[...] = jnp.broadcast_to(sum_new, sum_s.shape)
    sq_s[...] = jnp.broadcast_to(sq_new, sq_s.shape)

    nf = float(T)
    var = (sq_new - sum_new * sum_new / nf) / (nf - 1.0)
    cnt_ref[...] = jnp.broadcast_to(newcnt, cnt_ref.shape)
    var_ref[...] = jnp.broadcast_to(var, var_ref.shape)


_gating_call = pl.pallas_call(
    _gating_body,
    grid=(NB_A,),
    in_specs=[
        pl.BlockSpec((TB, D), lambda i: (i, 0)),
        pl.BlockSpec((E, D), lambda i: (0, 0)),
        pl.BlockSpec((E, 128), lambda i: (0, 0)),
    ],
    out_specs=[
        pl.BlockSpec((1, 1, TB), lambda i: (i, 0, 0)),
        pl.BlockSpec((1, 1, TB), lambda i: (i, 0, 0)),
        pl.BlockSpec((E, 128), lambda i: (0, 0)),
        pl.BlockSpec((E, 128), lambda i: (0, 0)),
    ],
    out_shape=[
        jax.ShapeDtypeStruct((NB_A, 1, TB), jnp.int32),
        jax.ShapeDtypeStruct((NB_A, 1, TB), jnp.int32),
        jax.ShapeDtypeStruct((E, 128), jnp.float32),
        jax.ShapeDtypeStruct((E, 128), jnp.float32),
    ],
    scratch_shapes=[
        pltpu.VMEM((E, 128), jnp.float32),
        pltpu.VMEM((E, 128), jnp.float32),
        pltpu.VMEM((E, 128), jnp.float32),
        pltpu.VMEM((TB, TB), jnp.float32),
    ],
)


_NC, _NS = 2, 16
_NW = _NC * _NS
_TPW = T // _NW
_CH = 16
_NCH = _TPW // _CH
_NBUF = 8

_sc_mesh = plsc.VectorSubcoreMesh(core_axis_name="c", subcore_axis_name="s")


@functools.partial(
    pl.kernel,
    out_type=[
        jax.ShapeDtypeStruct((PAD, D), jnp.float32),
        jax.ShapeDtypeStruct((T // _CH, _CH), jnp.int32),
    ],
    mesh=_sc_mesh,
    scratch_types=[
        pltpu.VMEM((_TPW,), jnp.int32),
        pltpu.VMEM((_TPW,), jnp.int32),
        pltpu.VMEM((16,), jnp.int32),
        pltpu.VMEM((_NCH, _CH), jnp.int32),
        pltpu.VMEM((_NBUF, _CH, D), jnp.float32),
        pltpu.SemaphoreType.DMA,
        pltpu.SemaphoreType.DMA,
    ],
    compiler_params=pltpu.CompilerParams(needs_layout_passes=False),
)
def _dispatch_call(flat_hbm, idx_hbm, rank_hbm, base_hbm, xs_hbm, pos_hbm,
                   idx_v, rank_v, base_v, pos_c, rows, sem_lin, sem_ind):
    wid = lax.axis_index("s") * _NC + lax.axis_index("c")
    t0 = wid * _TPW
    lins = [None] * _NCH
    inds = [None] * _NCH
    for j in range(_NBUF - 1):
        lins[j] = pltpu.async_copy(flat_hbm.at[pl.ds(t0 + j * _CH, _CH)],
                                   rows.at[j], sem_lin)
    pltpu.sync_copy(idx_hbm.at[pl.ds(t0, _TPW)], idx_v)
    pltpu.sync_copy(rank_hbm.at[pl.ds(t0, _TPW)], rank_v)
    pltpu.sync_copy(base_hbm, base_v)
    for c in range(_TPW // 16):
        e16 = idx_v[pl.ds(c * 16, 16)]
        b16 = plsc.load_gather(base_v, [e16])
        r16 = rank_v[pl.ds(c * 16, 16)]
        pos_c[c * 16 // _CH, pl.ds((c * 16) % _CH, 16)] = b16 + r16
    pltpu.sync_copy(pos_c, pos_hbm.at[pl.ds(wid * _NCH, _NCH)])
    for j in range(_NCH):
        k = j + _NBUF - 1
        if k < _NCH:
            if j >= 1:
                inds[j - 1].wait()
            lins[k] = pltpu.async_copy(
                flat_hbm.at[pl.ds(t0 + k * _CH, _CH)],
                rows.at[k % _NBUF], sem_lin)
        lins[j].wait()
        inds[j] = pltpu.async_copy(rows.at[j % _NBUF],
                                   xs_hbm.at[pos_c.at[j]], sem_ind)
    for j in range(max(0, _NCH - _NBUF), _NCH):
        inds[j].wait()


@functools.partial(
    pl.kernel,
    out_type=jax.ShapeDtypeStruct((T, D), jnp.float32),
    mesh=_sc_mesh,
    scratch_types=[
        pltpu.VMEM((_NCH, _CH), jnp.int32),
        pltpu.VMEM((_NBUF, _CH, D), jnp.float32),
        pltpu.SemaphoreType.DMA,
        pltpu.SemaphoreType.DMA,
    ],
)
def _combine_call(ys_hbm, pos_hbm, out_hbm, pos_c, rows, sem_g, sem_lin):
    wid = lax.axis_index("s") * _NC + lax.axis_index("c")
    t0 = wid * _TPW
    pltpu.sync_copy(pos_hbm.at[pl.ds(wid * _NCH, _NCH)], pos_c)
    gats = [None] * _NCH
    outs = [None] * _NCH
    for j in range(_NBUF - 1):
        gats[j] = pltpu.async_copy(ys_hbm.at[pos_c.at[j]], rows.at[j], sem_g)
    for j in range(_NCH):
        k = j + _NBUF - 1
        if k < _NCH:
            if j >= 1:
                outs[j - 1].wait()
            gats[k] = pltpu.async_copy(ys_hbm.at[pos_c.at[k]],
                                       rows.at[k % _NBUF], sem_g)
        gats[j].wait()
        outs[j] = pltpu.async_copy(rows.at[j % _NBUF],
                                   out_hbm.at[pl.ds(t0 + j * _CH, _CH)],
                                   sem_lin)
    for j in range(max(0, _NCH - _NBUF), _NCH):
        outs[j].wait()


def _matmul_body(be_ref, slot_ref, first_ref, pref_ref, act_ref, x_ref, w_hbm,
                 b_ref, o_ref, w2, sems):
    i = pl.program_id(0)
    slot = slot_ref[i]

    @pl.when(i == 0)
    def _():
        pltpu.make_async_copy(w_hbm.at[be_ref[0]], w2.at[0], sems.at[0]).start()

    @pl.when(pref_ref[i] == 1)
    def _():
        nslot = slot_ref[i + 1]
        pltpu.make_async_copy(w_hbm.at[be_ref[i + 1]], w2.at[nslot],
                              sems.at[nslot]).start()

    @pl.when(first_ref[i] == 1)
    def _():
        pltpu.make_async_copy(w_hbm.at[be_ref[i]], w2.at[slot],
                              sems.at[slot]).wait()

    for s in range(2):
        @pl.when(jnp.logical_and(act_ref[i] == 1, slot == s))
        def _(s=s):
            x = x_ref[...]
            o_ref[...] = jax.lax.dot_general(
                x, w2[s], (((1,), (1,)), ((), ())),
                preferred_element_type=jnp.float32) + b_ref[0]


def _act_idx(i, act):
    return jnp.where(act[i] == 1, i, NB_C - 1)


_matmul_call = pl.pallas_call(
    _matmul_body,
    grid_spec=pltpu.PrefetchScalarGridSpec(
        num_scalar_prefetch=5,
        grid=(NB_C,),
        in_specs=[
            pl.BlockSpec((BLK, D),
                         lambda i, be, sl, fi, pf, ac: (_act_idx(i, ac), 0)),
            pl.BlockSpec(memory_space=pl.ANY),
            pl.BlockSpec((1, 1, D),
                         lambda i, be, sl, fi, pf, ac: (be[i], 0, 0)),
        ],
        out_specs=pl.BlockSpec(
            (BLK, D), lambda i, be, sl, fi, pf, ac: (_act_idx(i, ac), 0)),
        scratch_shapes=[
            pltpu.VMEM((2, D, D), jnp.float32),
            pltpu.SemaphoreType.DMA((2,)),
        ],
    ),
    out_shape=jax.ShapeDtypeStruct((PAD, D), jnp.float32),
)


def kernel(sequences, expert_w, expert_b, gating_w, gating_b):
    flat = sequences.reshape(T, D)
    gb2 = jnp.broadcast_to(gating_b[:, None], (E, 128))

    idx3, rank3, cnt, var = _gating_call(flat, gating_w, gb2)
    idx = idx3.reshape(T)
    rank = rank3.reshape(T)
    variances = var[:, 0]

    counts = cnt[:, 0].astype(jnp.int32)
    padded = ((counts + BLK - 1) // BLK) * BLK
    cum = jnp.cumsum(padded)
    base = cum - padded
    blk_starts = jnp.arange(NB_C, dtype=jnp.int32) * BLK
    be = jnp.minimum(
        jnp.sum((cum[None, :] <= blk_starts[:, None]).astype(jnp.int32), axis=1),
        E - 1).astype(jnp.int32)

    base16 = jnp.concatenate([base, jnp.zeros((8,), jnp.int32)])

    nactive = cum[E - 1] // BLK
    act = (jnp.arange(NB_C, dtype=jnp.int32) < nactive).astype(jnp.int32)
    chg = (be[1:] != be[:-1]).astype(jnp.int32)
    first = jnp.concatenate([jnp.ones((1,), jnp.int32), chg]) * act
    slot = (jnp.cumsum(first) - 1) % 2
    pref = jnp.concatenate([first[1:], jnp.zeros((1,), jnp.int32)])

    x_sorted, pos = _dispatch_call(flat, idx, rank, base16)
    y_sorted = _matmul_call(be, slot, first, pref, act, x_sorted, expert_w,
                            expert_b.reshape(E, 1, D))
    out = _combine_call(y_sorted, pos)
    return (out.reshape(B, S, D), variances)
